# baseline (speedup 1.0000x reference)
import jax
import jax.numpy as jnp
from jax import lax
from jax.experimental import pallas as pl
from jax.experimental.pallas import tpu as pltpu
from jax.experimental.pallas import tpu_sc as plsc

_T, _N, _E = 8, 10000, 320000
_HDIM = 128
_NC, _NS = 2, 16
_NW = _NC * _NS
_K = 128
_CH = 80
_EPW = _CH * _K
_EPAD = _EPW * _NW
_NACC = 10240
_RPT = _NACC // _NS
_RBLK = 1280


def _sc_mesh():
    return plsc.VectorSubcoreMesh(
        core_axis_name="c", subcore_axis_name="s",
        num_cores=_NC, num_subcores=_NS)


def _make_segsum128():
    scratch = [
        pltpu.VMEM((2, 2, _K), jnp.int32),
        pltpu.VMEM((_K, 128), jnp.float32),
        pltpu.VMEM_SHARED((_NACC, 128), jnp.float32),
        pltpu.SemaphoreType.DMA,
        pltpu.SemaphoreType.DMA,
    ]

    def body(idx2_hbm, tab, zeros, out, idx_v, rows, acc, sem, isem):
        cid = lax.axis_index("c")
        tid = lax.axis_index("s")
        wid = tid * _NC + cid
        r0 = tid * _RPT
        cbase = wid * _CH
        pltpu.sync_copy(idx2_hbm.at[cbase], idx_v.at[0])
        pltpu.sync_copy(zeros, acc.at[pl.ds(r0, _RPT)])
        plsc.subcore_barrier()

        def pair(s, carry):
            for k in (0, 1):
                j = 2 * s + k

                @pl.when(j + 1 < _CH)
                def _():
                    pltpu.async_copy(idx2_hbm.at[cbase + j + 1],
                                     idx_v.at[1 - k], isem)

                pltpu.async_copy(tab.at[idx_v.at[k, 0]], rows, sem).wait()
                pltpu.sync_copy(rows, acc.at[idx_v.at[k, 1]], add=True)

                @pl.when(j + 1 < _CH)
                def _():
                    pltpu.make_async_copy(idx2_hbm.at[cbase],
                                          idx_v.at[1 - k], isem).wait()
            return carry

        lax.fori_loop(0, _CH // 2, pair, 0)
        plsc.subcore_barrier()
        pltpu.sync_copy(acc.at[pl.ds(r0, _RPT)],
                        out.at[pl.ds(cid * _NACC + r0, _RPT)])

    return pl.kernel(
        body,
        out_type=jax.ShapeDtypeStruct((2 * _NACC, 128), jnp.float32),
        mesh=_sc_mesh(),
        scratch_types=scratch,
        name="segsum128",
    )


def _make_degree():
    scratch = [
        pltpu.VMEM((_K,), jnp.int32),
        pltpu.VMEM((_K, 128), jnp.float32),
        pltpu.VMEM_SHARED((_NACC, 128), jnp.float32),
        pltpu.SemaphoreType.DMA,
    ]

    def body(idx2_hbm, ones_hbm, zeros_hbm, out, src_v, ones_v, acc, sem):
        cid = lax.axis_index("c")
        tid = lax.axis_index("s")
        wid = tid * _NC + cid
        r0 = tid * _RPT
        pltpu.sync_copy(zeros_hbm, acc.at[pl.ds(r0, _RPT)])
        pltpu.sync_copy(ones_hbm, ones_v)
        plsc.subcore_barrier()
        cbase = wid * _CH

        def chunk(j, carry):
            pltpu.sync_copy(idx2_hbm.at[cbase + j, 0], src_v)
            pltpu.sync_copy(ones_v, acc.at[src_v], add=True)
            return carry

        lax.fori_loop(0, _CH, chunk, 0)
        plsc.subcore_barrier()
        pltpu.sync_copy(acc.at[pl.ds(r0, _RPT)],
                        out.at[pl.ds(cid * _NACC + r0, _RPT)])

    return pl.kernel(
        body,
        out_type=jax.ShapeDtypeStruct((2 * _NACC, 128), jnp.float32),
        mesh=_sc_mesh(),
        scratch_types=scratch,
        name="edge_degree",
    )


def _row_spec(d, blk=None):
    blk = blk if blk is not None else _RBLK
    return pl.BlockSpec((blk, d), lambda i: (i, 0))


def _pair_spec(d):
    return pl.BlockSpec((2, _RBLK, d), lambda i: (0, i, 0))


def _full_spec(shape):
    return pl.BlockSpec(shape, lambda i: tuple(0 for _ in shape))


def _tc0_body(degp_ref, apack_ref, dis_ref, st_ref):
    deg = degp_ref[0, :, 0:1] + degp_ref[1, :, 0:1]
    row = (pl.program_id(0) * _RBLK
           + lax.broadcasted_iota(jnp.int32, (_RBLK, 1), 0))
    dis = jnp.where(deg > 0, lax.rsqrt(jnp.maximum(deg, 1e-12)), 0.0)
    dis = jnp.where(row < _N, dis, 0.0)
    dis_ref[...] = dis
    st_ref[...] = dis * apack_ref[...]


def _tc0(degp, apack):
    return pl.pallas_call(
        _tc0_body,
        grid=(_NACC // _RBLK,),
        in_specs=[_pair_spec(128), _row_spec(128)],
        out_specs=[_row_spec(1), _row_spec(128)],
        out_shape=[jax.ShapeDtypeStruct((_NACC, 1), jnp.float32),
                   jax.ShapeDtypeStruct((_NACC, 128), jnp.float32)],
        name="tc0_degnorm",
    )(degp, apack)


def _tca_body(xt_ref, h_ref, shp_ref, sap_ref, dgp_ref, dis_ref, wxuv_ref,
              wx0_ref, wx1_ref, bx_ref, wh3_ref, b3_ref,
              whzr0_ref, whzr1_ref, bzr_ref, whh0_ref, bhh_ref,
              z_ref, p_ref, gs_ref):
    x = xt_ref[...]
    h = h_ref[...]
    d = dis_ref[...]
    sh = shp_ref[0] + shp_ref[1]
    sa = sap_ref[0] + sap_ref[1]
    degw = dgp_ref[0] + dgp_ref[1]
    su = (jnp.dot(sh, wh3_ref[...], preferred_element_type=jnp.float32)
          + degw * b3_ref[...])
    convx = (jnp.dot(x, wx0_ref[...], preferred_element_type=jnp.float32)
             + jnp.dot(-d * sa, wx1_ref[...],
                       preferred_element_type=jnp.float32)
             + jnp.dot(-d * su, wxuv_ref[...],
                       preferred_element_type=jnp.float32)
             + bx_ref[...])
    convh = (jnp.dot(h, whzr0_ref[...], preferred_element_type=jnp.float32)
             + jnp.dot(-d * sh, whzr1_ref[...],
                       preferred_element_type=jnp.float32) + bzr_ref[...])
    z = jax.nn.sigmoid(convx[:, :128] + convh[:, :128])
    r = jax.nn.sigmoid(convx[:, 128:256] + convh[:, 128:])
    g = h * r
    z_ref[...] = z
    gs_ref[...] = d * g
    p_ref[...] = (convx[:, 256:]
                  + jnp.dot(g, whh0_ref[...],
                            preferred_element_type=jnp.float32) + bhh_ref[...])


def _tca(xt, h, shp, sap, dgp, dis, wxuv, w):
    return pl.pallas_call(
        _tca_body,
        grid=(_NACC // _RBLK,),
        in_specs=[_row_spec(16), _row_spec(128), _pair_spec(128),
                  _pair_spec(16), _pair_spec(1), _row_spec(1),
                  _full_spec((8, 384)),
                  _full_spec((16, 384)), _full_spec((16, 384)),
                  _full_spec((1, 384)), _full_spec((128, 8)),
                  _full_spec((1, 8)),
                  _full_spec((128, 256)), _full_spec((128, 256)),
                  _full_spec((1, 256)),
                  _full_spec((128, 128)), _full_spec((1, 128))],
        out_specs=[_row_spec(128), _row_spec(128), _row_spec(128)],
        out_shape=[jax.ShapeDtypeStruct((_NACC, 128), jnp.float32)] * 3,
        name="tca_gates",
    )(xt, h, shp, sap, dgp, dis, wxuv,
      w["wx0"], w["wx1"], w["bx"], w["wh3"], w["b3"],
      w["whzr0"], w["whzr1"], w["bzr"], w["whh0"], w["bhh"])


def _tcb_body(z_ref, p_ref, h_ref, sgp_ref, dis_ref, xto_ref, xp1_ref,
              whh1_ref, whead_ref, bhead_ref,
              hn_ref, tabh_ref, xn_ref, u_ref):
    d = dis_ref[...]
    sg = sgp_ref[0] + sgp_ref[1]
    ht = jnp.tanh(p_ref[...]
                  + jnp.dot(-d * sg, whh1_ref[...],
                            preferred_element_type=jnp.float32))
    z = z_ref[...]
    h = h_ref[...]
    hn = z * h + (1.0 - z) * ht
    u = jnp.dot(hn, whead_ref[...],
                preferred_element_type=jnp.float32) + bhead_ref[...]
    hn_ref[...] = hn
    tabh_ref[...] = d * hn
    u_ref[...] = u
    xto = xto_ref[...]
    xp1 = xp1_ref[...]
    dt = xp1[:, 6:7] - xto[:, 6:7]
    u3 = u[:, :3]
    v3 = (u3 - xto[:, 3:6]) / jnp.where(dt == 0, 1.0, dt)
    xn = jnp.concatenate(
        [xp1[:, :3], u3, xp1[:, 6:8], v3,
         jnp.zeros((_RBLK, 5), jnp.float32)], axis=1)
    xn_ref[...] = xn


def _tcb(z, p, h, sgp, dis, xto, xp1, w):
    return pl.pallas_call(
        _tcb_body,
        grid=(_NACC // _RBLK,),
        in_specs=[_row_spec(128), _row_spec(128), _row_spec(128),
                  _pair_spec(128), _row_spec(1), _row_spec(16),
                  _row_spec(16),
                  _full_spec((128, 128)), _full_spec((128, 8)),
                  _full_spec((1, 8))],
        out_specs=[_row_spec(128), _row_spec(128), _row_spec(16),
                   _row_spec(8)],
        out_shape=[jax.ShapeDtypeStruct((_NACC, 128), jnp.float32),
                   jax.ShapeDtypeStruct((_NACC, 128), jnp.float32),
                   jax.ShapeDtypeStruct((_NACC, 16), jnp.float32),
                   jax.ShapeDtypeStruct((_NACC, 8), jnp.float32)],
        name="tcb_update",
    )(z, p, h, sgp, dis, xto, xp1,
      w["whh1"], w["whead"], w["bhead"])


def kernel(X_seq, edge, Wxz0, Wxz1, bxz, Whz0, Whz1, bhz, Wxr0, Wxr1, bxr,
           Whr0, Whr1, bhr, Wxh0, Wxh1, bxh, Whh0, Whh1, bhh,
           W_head, b_head):
    f32 = jnp.float32

    src = edge[0].astype(jnp.int32)
    dst = edge[1].astype(jnp.int32)
    srcp = jnp.where(src == dst, _N, src)
    pad = _EPAD - _E
    srcp = jnp.concatenate([srcp, jnp.full((pad,), _N, jnp.int32)])
    dstp = jnp.concatenate([dst, jnp.full((pad,), _N, jnp.int32)])
    idx2 = jnp.stack([srcp.reshape(-1, _K), dstp.reshape(-1, _K)], axis=1)

    w = {
        "wx0": jnp.pad(jnp.concatenate([Wxz0, Wxr0, Wxh0], 0).T,
                       ((0, 5), (0, 0))),
        "wx1": jnp.pad(jnp.concatenate([Wxz1, Wxr1, Wxh1], 0).T,
                       ((0, 5), (0, 0))),
        "bx": jnp.concatenate([bxz, bxr, bxh])[None, :],
        "whzr0": jnp.concatenate([Whz0, Whr0], 0).T,
        "whzr1": jnp.concatenate([Whz1, Whr1], 0).T,
        "bzr": jnp.concatenate([bhz, bhr])[None, :],
        "whh0": Whh0.T,
        "bhh": bhh[None, :],
        "whh1": Whh1.T,
        "whead": jnp.pad(W_head, ((0, 5), (0, 0))).T,
        "bhead": jnp.pad(b_head, (0, 5))[None, :],
        "wh3": jnp.pad(W_head, ((0, 5), (0, 0))).T,
        "b3": jnp.pad(b_head, (0, 5))[None, :],
    }

    zeros128 = jnp.zeros((_RPT, 128), f32)
    ones128 = jnp.ones((_K, 128), f32)
    zparts128 = jnp.zeros((2, _NACC, 128), f32)
    h0 = jnp.zeros((_NACC, 128), f32)

    def pad_x(x):
        return jnp.pad(x, ((0, _NACC - _N), (0, 5)))

    dts = [None] + [X_seq[t, 0, 6] - X_seq[t - 1, 0, 6] for t in range(1, _T)]
    slots = [pad_x(X_seq[0]).at[:, 11].set(1.0)]
    for t in range(1, _T):
        a_t = pad_x(X_seq[t - 1])[:, 3:6]
        at = pad_x(X_seq[t])
        at = at.at[:, 3:6].set(0.0)
        at = at.at[:, 8:11].set(-a_t / dts[t])
        slots.append(at)
    apack = jnp.concatenate(slots, axis=1)

    eu = jnp.zeros((3, 16), f32).at[jnp.arange(3), jnp.arange(3, 6)].set(1.0)
    ev = jnp.zeros((3, 16), f32).at[jnp.arange(3), jnp.arange(8, 11)].set(1.0)
    wxuv = [jnp.zeros((8, 384), f32)]
    for t in range(1, _T):
        m = (eu + ev / dts[t]) @ w["wx1"]
        wxuv.append(jnp.pad(m, ((0, 5), (0, 0))))

    degree = _make_degree()
    segsum = _make_segsum128()

    degp = degree(idx2, ones128, zeros128)
    dis, st_tab = _tc0(degp.reshape(2, _NACC, 128), apack)
    sstp = segsum(idx2, st_tab, zeros128).reshape(2, _NACC, 128)
    sap = [sstp[:, :, 16 * t:16 * (t + 1)] for t in range(_T)]
    dgp = sstp[:, :, 11:12]

    xt = pad_x(X_seq[0])
    h = h0
    tabh = None
    outs = []
    for t in range(_T):
        if t == 0:
            shp = zparts128
        else:
            shp = segsum(idx2, tabh, zeros128).reshape(2, _NACC, 128)
        z, p, gs = _tca(xt, h, shp, sap[t], dgp, dis, wxuv[t], w)
        if t == 0:
            sgp = zparts128
        else:
            sgp = segsum(idx2, gs, zeros128).reshape(2, _NACC, 128)
        xto = pad_x(X_seq[t])
        xp1 = pad_x(X_seq[t + 1]) if t < _T - 1 else xto
        h, tabh, xn, u8 = _tcb(z, p, h, sgp, dis, xto, xp1, w)
        outs.append(u8[:_N, :3])
        xt = xn
    return jnp.stack(outs)


# speedup vs baseline: 1.2754x; 1.2754x over previous
import jax
import jax.numpy as jnp
from jax import lax
from jax.experimental import pallas as pl
from jax.experimental.pallas import tpu as pltpu
from jax.experimental.pallas import tpu_sc as plsc

_T, _N, _E = 8, 10000, 320000
_HDIM = 128
_NC, _NS = 2, 16
_NW = _NC * _NS
_K = 128
_CH = -(-_E // (_NW * _K))
_EPW = _CH * _K
_EPAD = _EPW * _NW
_NACC = 10240
_RPT = _NACC // _NS
_RBLK = 1280


def _sc_mesh():
    return plsc.VectorSubcoreMesh(
        core_axis_name="c", subcore_axis_name="s",
        num_cores=_NC, num_subcores=_NS)


def _make_segsum128():
    scratch = [
        pltpu.VMEM((2, _K), jnp.int32),
        pltpu.VMEM((_K, 128), jnp.float32),
        pltpu.VMEM_SHARED((_NACC, 128), jnp.float32),
        pltpu.SemaphoreType.DMA,
    ]

    def body(idx2_hbm, tab, zeros, out, idx_v, rows, acc, sem):
        cid = lax.axis_index("c")
        tid = lax.axis_index("s")
        wid = tid * _NC + cid
        r0 = tid * _RPT
        pltpu.sync_copy(zeros, acc.at[pl.ds(r0, _RPT)])
        plsc.subcore_barrier()
        cbase = wid * _CH

        def chunk(j, carry):
            pltpu.sync_copy(idx2_hbm.at[cbase + j], idx_v)
            pltpu.async_copy(tab.at[idx_v.at[0]], rows, sem).wait()
            pltpu.sync_copy(rows, acc.at[idx_v.at[1]], add=True)
            return carry

        lax.fori_loop(0, _CH, chunk, 0)
        plsc.subcore_barrier()
        pltpu.sync_copy(acc.at[pl.ds(r0, _RPT)],
                        out.at[pl.ds(cid * _NACC + r0, _RPT)])

    return pl.kernel(
        body,
        out_type=jax.ShapeDtypeStruct((2 * _NACC, 128), jnp.float32),
        mesh=_sc_mesh(),
        scratch_types=scratch,
        name="segsum128",
    )


def _make_degree():
    scratch = [
        pltpu.VMEM((_K,), jnp.int32),
        pltpu.VMEM((_K, 128), jnp.float32),
        pltpu.VMEM_SHARED((_NACC, 128), jnp.float32),
        pltpu.SemaphoreType.DMA,
    ]

    def body(idx2_hbm, ones_hbm, zeros_hbm, out, src_v, ones_v, acc, sem):
        cid = lax.axis_index("c")
        tid = lax.axis_index("s")
        wid = tid * _NC + cid
        r0 = tid * _RPT
        pltpu.sync_copy(zeros_hbm, acc.at[pl.ds(r0, _RPT)])
        pltpu.sync_copy(ones_hbm, ones_v)
        plsc.subcore_barrier()
        cbase = wid * _CH

        def chunk(j, carry):
            pltpu.sync_copy(idx2_hbm.at[cbase + j, 0], src_v)
            pltpu.sync_copy(ones_v, acc.at[src_v], add=True)
            return carry

        lax.fori_loop(0, _CH, chunk, 0)
        plsc.subcore_barrier()
        pltpu.sync_copy(acc.at[pl.ds(r0, _RPT)],
                        out.at[pl.ds(cid * _NACC + r0, _RPT)])

    return pl.kernel(
        body,
        out_type=jax.ShapeDtypeStruct((2 * _NACC, 128), jnp.float32),
        mesh=_sc_mesh(),
        scratch_types=scratch,
        name="edge_degree",
    )


def _row_spec(d, blk=None):
    blk = blk if blk is not None else _RBLK
    return pl.BlockSpec((blk, d), lambda i: (i, 0))


def _pair_spec(d):
    return pl.BlockSpec((2, _RBLK, d), lambda i: (0, i, 0))


def _full_spec(shape):
    return pl.BlockSpec(shape, lambda i: tuple(0 for _ in shape))


def _tc0_body(degp_ref, apack_ref, dis_ref, st_ref):
    deg = degp_ref[0, :, 0:1] + degp_ref[1, :, 0:1]
    row = (pl.program_id(0) * _RBLK
           + lax.broadcasted_iota(jnp.int32, (_RBLK, 1), 0))
    dis = jnp.where(deg > 0, lax.rsqrt(jnp.maximum(deg, 1e-12)), 0.0)
    dis = jnp.where(row < _N, dis, 0.0)
    dis_ref[...] = dis
    st_ref[...] = dis * apack_ref[...]


def _tc0(degp, apack):
    return pl.pallas_call(
        _tc0_body,
        grid=(_NACC // _RBLK,),
        in_specs=[_pair_spec(128), _row_spec(128)],
        out_specs=[_row_spec(1), _row_spec(128)],
        out_shape=[jax.ShapeDtypeStruct((_NACC, 1), jnp.float32),
                   jax.ShapeDtypeStruct((_NACC, 128), jnp.float32)],
        name="tc0_degnorm",
    )(degp, apack)


def _tca_body(xt_ref, h_ref, shp_ref, sap_ref, dgp_ref, dis_ref, wxuv_ref,
              wx0_ref, wx1_ref, bx_ref, wh3_ref, b3_ref,
              whzr0_ref, whzr1_ref, bzr_ref, whh0_ref, bhh_ref,
              z_ref, p_ref, gs_ref):
    x = xt_ref[...]
    h = h_ref[...]
    d = dis_ref[...]
    sh = shp_ref[0] + shp_ref[1]
    sa = sap_ref[0] + sap_ref[1]
    degw = dgp_ref[0] + dgp_ref[1]
    su = (jnp.dot(sh, wh3_ref[...], preferred_element_type=jnp.float32)
          + degw * b3_ref[...])
    convx = (jnp.dot(x, wx0_ref[...], preferred_element_type=jnp.float32)
             + jnp.dot(-d * sa, wx1_ref[...],
                       preferred_element_type=jnp.float32)
             + jnp.dot(-d * su, wxuv_ref[...],
                       preferred_element_type=jnp.float32)
             + bx_ref[...])
    convh = (jnp.dot(h, whzr0_ref[...], preferred_element_type=jnp.float32)
             + jnp.dot(-d * sh, whzr1_ref[...],
                       preferred_element_type=jnp.float32) + bzr_ref[...])
    z = jax.nn.sigmoid(convx[:, :128] + convh[:, :128])
    r = jax.nn.sigmoid(convx[:, 128:256] + convh[:, 128:])
    g = h * r
    z_ref[...] = z
    gs_ref[...] = d * g
    p_ref[...] = (convx[:, 256:]
                  + jnp.dot(g, whh0_ref[...],
                            preferred_element_type=jnp.float32) + bhh_ref[...])


def _tca(xt, h, shp, sap, dgp, dis, wxuv, w):
    return pl.pallas_call(
        _tca_body,
        grid=(_NACC // _RBLK,),
        in_specs=[_row_spec(16), _row_spec(128), _pair_spec(128),
                  _pair_spec(16), _pair_spec(1), _row_spec(1),
                  _full_spec((8, 384)),
                  _full_spec((16, 384)), _full_spec((16, 384)),
                  _full_spec((1, 384)), _full_spec((128, 8)),
                  _full_spec((1, 8)),
                  _full_spec((128, 256)), _full_spec((128, 256)),
                  _full_spec((1, 256)),
                  _full_spec((128, 128)), _full_spec((1, 128))],
        out_specs=[_row_spec(128), _row_spec(128), _row_spec(128)],
        out_shape=[jax.ShapeDtypeStruct((_NACC, 128), jnp.float32)] * 3,
        name="tca_gates",
    )(xt, h, shp, sap, dgp, dis, wxuv,
      w["wx0"], w["wx1"], w["bx"], w["wh3"], w["b3"],
      w["whzr0"], w["whzr1"], w["bzr"], w["whh0"], w["bhh"])


def _tcb_body(z_ref, p_ref, h_ref, sgp_ref, dis_ref, xto_ref, xp1_ref,
              whh1_ref, whead_ref, bhead_ref,
              hn_ref, tabh_ref, xn_ref, u_ref):
    d = dis_ref[...]
    sg = sgp_ref[0] + sgp_ref[1]
    ht = jnp.tanh(p_ref[...]
                  + jnp.dot(-d * sg, whh1_ref[...],
                            preferred_element_type=jnp.float32))
    z = z_ref[...]
    h = h_ref[...]
    hn = z * h + (1.0 - z) * ht
    u = jnp.dot(hn, whead_ref[...],
                preferred_element_type=jnp.float32) + bhead_ref[...]
    hn_ref[...] = hn
    tabh_ref[...] = d * hn
    u_ref[...] = u
    xto = xto_ref[...]
    xp1 = xp1_ref[...]
    dt = xp1[:, 6:7] - xto[:, 6:7]
    u3 = u[:, :3]
    v3 = (u3 - xto[:, 3:6]) / jnp.where(dt == 0, 1.0, dt)
    xn = jnp.concatenate(
        [xp1[:, :3], u3, xp1[:, 6:8], v3,
         jnp.zeros((_RBLK, 5), jnp.float32)], axis=1)
    xn_ref[...] = xn


def _tcb(z, p, h, sgp, dis, xto, xp1, w):
    return pl.pallas_call(
        _tcb_body,
        grid=(_NACC // _RBLK,),
        in_specs=[_row_spec(128), _row_spec(128), _row_spec(128),
                  _pair_spec(128), _row_spec(1), _row_spec(16),
                  _row_spec(16),
                  _full_spec((128, 128)), _full_spec((128, 8)),
                  _full_spec((1, 8))],
        out_specs=[_row_spec(128), _row_spec(128), _row_spec(16),
                   _row_spec(8)],
        out_shape=[jax.ShapeDtypeStruct((_NACC, 128), jnp.float32),
                   jax.ShapeDtypeStruct((_NACC, 128), jnp.float32),
                   jax.ShapeDtypeStruct((_NACC, 16), jnp.float32),
                   jax.ShapeDtypeStruct((_NACC, 8), jnp.float32)],
        name="tcb_update",
    )(z, p, h, sgp, dis, xto, xp1,
      w["whh1"], w["whead"], w["bhead"])


def kernel(X_seq, edge, Wxz0, Wxz1, bxz, Whz0, Whz1, bhz, Wxr0, Wxr1, bxr,
           Whr0, Whr1, bhr, Wxh0, Wxh1, bxh, Whh0, Whh1, bhh,
           W_head, b_head):
    f32 = jnp.float32

    src = edge[0].astype(jnp.int32)
    dst = edge[1].astype(jnp.int32)
    srcp = jnp.where(src == dst, _N, src)
    pad = _EPAD - _E
    srcp = jnp.concatenate([srcp, jnp.full((pad,), _N, jnp.int32)])
    dstp = jnp.concatenate([dst, jnp.full((pad,), _N, jnp.int32)])
    idx2 = jnp.stack([srcp.reshape(-1, _K), dstp.reshape(-1, _K)], axis=1)

    w = {
        "wx0": jnp.pad(jnp.concatenate([Wxz0, Wxr0, Wxh0], 0).T,
                       ((0, 5), (0, 0))),
        "wx1": jnp.pad(jnp.concatenate([Wxz1, Wxr1, Wxh1], 0).T,
                       ((0, 5), (0, 0))),
        "bx": jnp.concatenate([bxz, bxr, bxh])[None, :],
        "whzr0": jnp.concatenate([Whz0, Whr0], 0).T,
        "whzr1": jnp.concatenate([Whz1, Whr1], 0).T,
        "bzr": jnp.concatenate([bhz, bhr])[None, :],
        "whh0": Whh0.T,
        "bhh": bhh[None, :],
        "whh1": Whh1.T,
        "whead": jnp.pad(W_head, ((0, 5), (0, 0))).T,
        "bhead": jnp.pad(b_head, (0, 5))[None, :],
        "wh3": jnp.pad(W_head, ((0, 5), (0, 0))).T,
        "b3": jnp.pad(b_head, (0, 5))[None, :],
    }

    zeros128 = jnp.zeros((_RPT, 128), f32)
    ones128 = jnp.ones((_K, 128), f32)
    zparts128 = jnp.zeros((2, _NACC, 128), f32)
    h0 = jnp.zeros((_NACC, 128), f32)

    def pad_x(x):
        return jnp.pad(x, ((0, _NACC - _N), (0, 5)))

    dts = [None] + [X_seq[t, 0, 6] - X_seq[t - 1, 0, 6] for t in range(1, _T)]
    slots = [pad_x(X_seq[0]).at[:, 11].set(1.0)]
    for t in range(1, _T):
        a_t = pad_x(X_seq[t - 1])[:, 3:6]
        at = pad_x(X_seq[t])
        at = at.at[:, 3:6].set(0.0)
        at = at.at[:, 8:11].set(-a_t / dts[t])
        slots.append(at)
    apack = jnp.concatenate(slots, axis=1)

    eu = jnp.zeros((3, 16), f32).at[jnp.arange(3), jnp.arange(3, 6)].set(1.0)
    ev = jnp.zeros((3, 16), f32).at[jnp.arange(3), jnp.arange(8, 11)].set(1.0)
    wxuv = [jnp.zeros((8, 384), f32)]
    for t in range(1, _T):
        m = (eu + ev / dts[t]) @ w["wx1"]
        wxuv.append(jnp.pad(m, ((0, 5), (0, 0))))

    degree = _make_degree()
    segsum = _make_segsum128()

    degp = degree(idx2, ones128, zeros128)
    dis, st_tab = _tc0(degp.reshape(2, _NACC, 128), apack)
    sstp = segsum(idx2, st_tab, zeros128).reshape(2, _NACC, 128)
    sap = [sstp[:, :, 16 * t:16 * (t + 1)] for t in range(_T)]
    dgp = sstp[:, :, 11:12]

    xt = pad_x(X_seq[0])
    h = h0
    tabh = None
    outs = []
    for t in range(_T):
        if t == 0:
            shp = zparts128
        else:
            shp = segsum(idx2, tabh, zeros128).reshape(2, _NACC, 128)
        z, p, gs = _tca(xt, h, shp, sap[t], dgp, dis, wxuv[t], w)
        if t == 0:
            sgp = zparts128
        else:
            sgp = segsum(idx2, gs, zeros128).reshape(2, _NACC, 128)
        xto = pad_x(X_seq[t])
        xp1 = pad_x(X_seq[t + 1]) if t < _T - 1 else xto
        h, tabh, xn, u8 = _tcb(z, p, h, sgp, dis, xto, xp1, w)
        outs.append(u8[:_N, :3])
        xt = xn
    return jnp.stack(outs)


# speedup vs baseline: 1.3753x; 1.0783x over previous
import jax
import jax.numpy as jnp
from jax import lax
from jax.experimental import pallas as pl
from jax.experimental.pallas import tpu as pltpu
from jax.experimental.pallas import tpu_sc as plsc

_T, _N, _E = 8, 10000, 320000
_HDIM = 128
_NC, _NS = 2, 16
_NW = _NC * _NS
_K = 128
_CH = -(-_E // (_NW * _K))
_EPW = _CH * _K
_EPAD = _EPW * _NW
_NACC = 10240
_RPT = _NACC // _NS
_RBLK = 1280


def _sc_mesh():
    return plsc.VectorSubcoreMesh(
        core_axis_name="c", subcore_axis_name="s",
        num_cores=_NC, num_subcores=_NS)


def _make_segsum128():
    scratch = [
        pltpu.VMEM((_CH, 2, _K), jnp.int32),
        pltpu.VMEM((_K, 128), jnp.float32),
        pltpu.VMEM_SHARED((_NACC, 128), jnp.float32),
        pltpu.SemaphoreType.DMA,
    ]

    def body(idx2_hbm, tab, zeros, out, idx_v, rows, acc, sem):
        cid = lax.axis_index("c")
        tid = lax.axis_index("s")
        wid = tid * _NC + cid
        r0 = tid * _RPT
        cbase = wid * _CH
        pltpu.sync_copy(idx2_hbm.at[pl.ds(cbase, _CH)], idx_v)
        pltpu.sync_copy(zeros, acc.at[pl.ds(r0, _RPT)])
        plsc.subcore_barrier()

        def chunk(j, carry):
            pltpu.async_copy(tab.at[idx_v.at[j, 0]], rows, sem).wait()
            pltpu.sync_copy(rows, acc.at[idx_v.at[j, 1]], add=True)
            return carry

        lax.fori_loop(0, _CH, chunk, 0)
        plsc.subcore_barrier()
        pltpu.sync_copy(acc.at[pl.ds(r0, _RPT)],
                        out.at[pl.ds(cid * _NACC + r0, _RPT)])

    return pl.kernel(
        body,
        out_type=jax.ShapeDtypeStruct((2 * _NACC, 128), jnp.float32),
        mesh=_sc_mesh(),
        scratch_types=scratch,
        name="segsum128",
    )


def _make_degree():
    scratch = [
        pltpu.VMEM((_CH, 2, _K), jnp.int32),
        pltpu.VMEM((_K, 128), jnp.float32),
        pltpu.VMEM_SHARED((_NACC, 128), jnp.float32),
        pltpu.SemaphoreType.DMA,
    ]

    def body(idx2_hbm, ones_hbm, zeros_hbm, out, idx_v, ones_v, acc, sem):
        cid = lax.axis_index("c")
        tid = lax.axis_index("s")
        wid = tid * _NC + cid
        r0 = tid * _RPT
        cbase = wid * _CH
        pltpu.sync_copy(idx2_hbm.at[pl.ds(cbase, _CH)], idx_v)
        pltpu.sync_copy(zeros_hbm, acc.at[pl.ds(r0, _RPT)])
        pltpu.sync_copy(ones_hbm, ones_v)
        plsc.subcore_barrier()

        def chunk(j, carry):
            pltpu.sync_copy(ones_v, acc.at[idx_v.at[j, 0]], add=True)
            return carry

        lax.fori_loop(0, _CH, chunk, 0)
        plsc.subcore_barrier()
        pltpu.sync_copy(acc.at[pl.ds(r0, _RPT)],
                        out.at[pl.ds(cid * _NACC + r0, _RPT)])

    return pl.kernel(
        body,
        out_type=jax.ShapeDtypeStruct((2 * _NACC, 128), jnp.float32),
        mesh=_sc_mesh(),
        scratch_types=scratch,
        name="edge_degree",
    )


def _row_spec(d, blk=None):
    blk = blk if blk is not None else _RBLK
    return pl.BlockSpec((blk, d), lambda i: (i, 0))


def _pair_spec(d):
    return pl.BlockSpec((2, _RBLK, d), lambda i: (0, i, 0))


def _full_spec(shape):
    return pl.BlockSpec(shape, lambda i: tuple(0 for _ in shape))


def _tc0_body(degp_ref, apack_ref, dis_ref, st_ref):
    deg = degp_ref[0, :, 0:1] + degp_ref[1, :, 0:1]
    row = (pl.program_id(0) * _RBLK
           + lax.broadcasted_iota(jnp.int32, (_RBLK, 1), 0))
    dis = jnp.where(deg > 0, lax.rsqrt(jnp.maximum(deg, 1e-12)), 0.0)
    dis = jnp.where(row < _N, dis, 0.0)
    dis_ref[...] = dis
    st_ref[...] = dis * apack_ref[...]


def _tc0(degp, apack):
    return pl.pallas_call(
        _tc0_body,
        grid=(_NACC // _RBLK,),
        in_specs=[_pair_spec(128), _row_spec(128)],
        out_specs=[_row_spec(1), _row_spec(128)],
        out_shape=[jax.ShapeDtypeStruct((_NACC, 1), jnp.float32),
                   jax.ShapeDtypeStruct((_NACC, 128), jnp.float32)],
        name="tc0_degnorm",
    )(degp, apack)


def _tca_body(xt_ref, h_ref, shp_ref, sap_ref, dgp_ref, dis_ref, wxuv_ref,
              wx0_ref, wx1_ref, bx_ref, wh3_ref, b3_ref,
              whzr0_ref, whzr1_ref, bzr_ref, whh0_ref, bhh_ref,
              z_ref, p_ref, gs_ref):
    x = xt_ref[...]
    h = h_ref[...]
    d = dis_ref[...]
    sh = shp_ref[0] + shp_ref[1]
    sa = sap_ref[0] + sap_ref[1]
    degw = dgp_ref[0] + dgp_ref[1]
    su = (jnp.dot(sh, wh3_ref[...], preferred_element_type=jnp.float32)
          + degw * b3_ref[...])
    convx = (jnp.dot(x, wx0_ref[...], preferred_element_type=jnp.float32)
             + jnp.dot(-d * sa, wx1_ref[...],
                       preferred_element_type=jnp.float32)
             + jnp.dot(-d * su, wxuv_ref[...],
                       preferred_element_type=jnp.float32)
             + bx_ref[...])
    convh = (jnp.dot(h, whzr0_ref[...], preferred_element_type=jnp.float32)
             + jnp.dot(-d * sh, whzr1_ref[...],
                       preferred_element_type=jnp.float32) + bzr_ref[...])
    z = jax.nn.sigmoid(convx[:, :128] + convh[:, :128])
    r = jax.nn.sigmoid(convx[:, 128:256] + convh[:, 128:])
    g = h * r
    z_ref[...] = z
    gs_ref[...] = d * g
    p_ref[...] = (convx[:, 256:]
                  + jnp.dot(g, whh0_ref[...],
                            preferred_element_type=jnp.float32) + bhh_ref[...])


def _tca(xt, h, shp, sap, dgp, dis, wxuv, w):
    return pl.pallas_call(
        _tca_body,
        grid=(_NACC // _RBLK,),
        in_specs=[_row_spec(16), _row_spec(128), _pair_spec(128),
                  _pair_spec(16), _pair_spec(1), _row_spec(1),
                  _full_spec((8, 384)),
                  _full_spec((16, 384)), _full_spec((16, 384)),
                  _full_spec((1, 384)), _full_spec((128, 8)),
                  _full_spec((1, 8)),
                  _full_spec((128, 256)), _full_spec((128, 256)),
                  _full_spec((1, 256)),
                  _full_spec((128, 128)), _full_spec((1, 128))],
        out_specs=[_row_spec(128), _row_spec(128), _row_spec(128)],
        out_shape=[jax.ShapeDtypeStruct((_NACC, 128), jnp.float32)] * 3,
        name="tca_gates",
    )(xt, h, shp, sap, dgp, dis, wxuv,
      w["wx0"], w["wx1"], w["bx"], w["wh3"], w["b3"],
      w["whzr0"], w["whzr1"], w["bzr"], w["whh0"], w["bhh"])


def _tcb_body(z_ref, p_ref, h_ref, sgp_ref, dis_ref, xto_ref, xp1_ref,
              whh1_ref, whead_ref, bhead_ref,
              hn_ref, tabh_ref, xn_ref, u_ref):
    d = dis_ref[...]
    sg = sgp_ref[0] + sgp_ref[1]
    ht = jnp.tanh(p_ref[...]
                  + jnp.dot(-d * sg, whh1_ref[...],
                            preferred_element_type=jnp.float32))
    z = z_ref[...]
    h = h_ref[...]
    hn = z * h + (1.0 - z) * ht
    u = jnp.dot(hn, whead_ref[...],
                preferred_element_type=jnp.float32) + bhead_ref[...]
    hn_ref[...] = hn
    tabh_ref[...] = d * hn
    u_ref[...] = u
    xto = xto_ref[...]
    xp1 = xp1_ref[...]
    dt = xp1[:, 6:7] - xto[:, 6:7]
    u3 = u[:, :3]
    v3 = (u3 - xto[:, 3:6]) / jnp.where(dt == 0, 1.0, dt)
    xn = jnp.concatenate(
        [xp1[:, :3], u3, xp1[:, 6:8], v3,
         jnp.zeros((_RBLK, 5), jnp.float32)], axis=1)
    xn_ref[...] = xn


def _tcb(z, p, h, sgp, dis, xto, xp1, w):
    return pl.pallas_call(
        _tcb_body,
        grid=(_NACC // _RBLK,),
        in_specs=[_row_spec(128), _row_spec(128), _row_spec(128),
                  _pair_spec(128), _row_spec(1), _row_spec(16),
                  _row_spec(16),
                  _full_spec((128, 128)), _full_spec((128, 8)),
                  _full_spec((1, 8))],
        out_specs=[_row_spec(128), _row_spec(128), _row_spec(16),
                   _row_spec(8)],
        out_shape=[jax.ShapeDtypeStruct((_NACC, 128), jnp.float32),
                   jax.ShapeDtypeStruct((_NACC, 128), jnp.float32),
                   jax.ShapeDtypeStruct((_NACC, 16), jnp.float32),
                   jax.ShapeDtypeStruct((_NACC, 8), jnp.float32)],
        name="tcb_update",
    )(z, p, h, sgp, dis, xto, xp1,
      w["whh1"], w["whead"], w["bhead"])


def kernel(X_seq, edge, Wxz0, Wxz1, bxz, Whz0, Whz1, bhz, Wxr0, Wxr1, bxr,
           Whr0, Whr1, bhr, Wxh0, Wxh1, bxh, Whh0, Whh1, bhh,
           W_head, b_head):
    f32 = jnp.float32

    src = edge[0].astype(jnp.int32)
    dst = edge[1].astype(jnp.int32)
    srcp = jnp.where(src == dst, _N, src)
    pad = _EPAD - _E
    srcp = jnp.concatenate([srcp, jnp.full((pad,), _N, jnp.int32)])
    dstp = jnp.concatenate([dst, jnp.full((pad,), _N, jnp.int32)])
    idx2 = jnp.stack([srcp.reshape(-1, _K), dstp.reshape(-1, _K)], axis=1)

    w = {
        "wx0": jnp.pad(jnp.concatenate([Wxz0, Wxr0, Wxh0], 0).T,
                       ((0, 5), (0, 0))),
        "wx1": jnp.pad(jnp.concatenate([Wxz1, Wxr1, Wxh1], 0).T,
                       ((0, 5), (0, 0))),
        "bx": jnp.concatenate([bxz, bxr, bxh])[None, :],
        "whzr0": jnp.concatenate([Whz0, Whr0], 0).T,
        "whzr1": jnp.concatenate([Whz1, Whr1], 0).T,
        "bzr": jnp.concatenate([bhz, bhr])[None, :],
        "whh0": Whh0.T,
        "bhh": bhh[None, :],
        "whh1": Whh1.T,
        "whead": jnp.pad(W_head, ((0, 5), (0, 0))).T,
        "bhead": jnp.pad(b_head, (0, 5))[None, :],
        "wh3": jnp.pad(W_head, ((0, 5), (0, 0))).T,
        "b3": jnp.pad(b_head, (0, 5))[None, :],
    }

    zeros128 = jnp.zeros((_RPT, 128), f32)
    ones128 = jnp.ones((_K, 128), f32)
    zparts128 = jnp.zeros((2, _NACC, 128), f32)
    h0 = jnp.zeros((_NACC, 128), f32)

    def pad_x(x):
        return jnp.pad(x, ((0, _NACC - _N), (0, 5)))

    dts = [None] + [X_seq[t, 0, 6] - X_seq[t - 1, 0, 6] for t in range(1, _T)]
    slots = [pad_x(X_seq[0]).at[:, 11].set(1.0)]
    for t in range(1, _T):
        a_t = pad_x(X_seq[t - 1])[:, 3:6]
        at = pad_x(X_seq[t])
        at = at.at[:, 3:6].set(0.0)
        at = at.at[:, 8:11].set(-a_t / dts[t])
        slots.append(at)
    apack = jnp.concatenate(slots, axis=1)

    eu = jnp.zeros((3, 16), f32).at[jnp.arange(3), jnp.arange(3, 6)].set(1.0)
    ev = jnp.zeros((3, 16), f32).at[jnp.arange(3), jnp.arange(8, 11)].set(1.0)
    wxuv = [jnp.zeros((8, 384), f32)]
    for t in range(1, _T):
        m = (eu + ev / dts[t]) @ w["wx1"]
        wxuv.append(jnp.pad(m, ((0, 5), (0, 0))))

    degree = _make_degree()
    segsum = _make_segsum128()

    degp = degree(idx2, ones128, zeros128)
    dis, st_tab = _tc0(degp.reshape(2, _NACC, 128), apack)
    sstp = segsum(idx2, st_tab, zeros128).reshape(2, _NACC, 128)
    sap = [sstp[:, :, 16 * t:16 * (t + 1)] for t in range(_T)]
    dgp = sstp[:, :, 11:12]

    xt = pad_x(X_seq[0])
    h = h0
    tabh = None
    outs = []
    for t in range(_T):
        if t == 0:
            shp = zparts128
        else:
            shp = segsum(idx2, tabh, zeros128).reshape(2, _NACC, 128)
        z, p, gs = _tca(xt, h, shp, sap[t], dgp, dis, wxuv[t], w)
        if t == 0:
            sgp = zparts128
        else:
            sgp = segsum(idx2, gs, zeros128).reshape(2, _NACC, 128)
        xto = pad_x(X_seq[t])
        xp1 = pad_x(X_seq[t + 1]) if t < _T - 1 else xto
        h, tabh, xn, u8 = _tcb(z, p, h, sgp, dis, xto, xp1, w)
        outs.append(u8[:_N, :3])
        xt = xn
    return jnp.stack(outs)
